# baseline (speedup 1.0000x reference)
import functools

import jax
import jax.numpy as jnp
from jax import lax
from jax.experimental import pallas as pl
from jax.experimental.pallas import tpu as pltpu
from jax.experimental.pallas import tpu_sc as plsc

_NC = 2
_NS = 16
_NW = _NC * _NS

_SHARD = 8192
_SEG = 1600
_CAP = _SHARD - 1
_CHUNK = 128


@functools.lru_cache(maxsize=None)
def _build_lookup(V, D, N):
    assert N % _NW == 0
    b_per_w = N // _NW
    assert b_per_w % _SEG == 0
    nseg = b_per_w // _SEG
    nshard = -(-V // _SHARD)
    n_full_pairs = nshard // 2
    last_rows = V - (nshard - 1) * _SHARD
    assert last_rows % _NS == 0
    max_pairs = -(-_SEG // (2 * _CHUNK)) + 1
    buf_len = max_pairs * 2 * _CHUNK

    mesh = plsc.VectorSubcoreMesh(
        core_axis_name="c", subcore_axis_name="s",
        num_cores=_NC, num_subcores=_NS,
    )

    @functools.partial(
        pl.kernel,
        mesh=mesh,
        out_type=jax.ShapeDtypeStruct((N, D), jnp.float32),
        scratch_types=[
            pltpu.VMEM((b_per_w,), jnp.int32),
            pltpu.VMEM((buf_len + 128,), jnp.int32),
            pltpu.VMEM((2, _CHUNK), jnp.int32),
            pltpu.VMEM((2, _CHUNK), jnp.int32),
            pltpu.VMEM((2, _CHUNK, D), jnp.float32),
            pltpu.VMEM_SHARED((_SHARD, D), jnp.float32),
            pltpu.SMEM((1,), jnp.int32),
            pltpu.SemaphoreType.DMA,
            [pltpu.SemaphoreType.DMA] * 2,
        ],
    )
    def k(table_hbm, idx_hbm, out_hbm, idx_v, packed_v, lrow_v, pos_v,
          rows_v, shard_sp, done_ref, gsem, ssem):
        cid = lax.axis_index("c")
        sid = lax.axis_index("s")
        wid = sid * _NC + cid
        base = wid * b_per_w
        lane = jnp.arange(16, dtype=jnp.int32)

        done_ref[0] = 0
        pltpu.sync_copy(idx_hbm.at[pl.ds(base, b_per_w)], idx_v)

        def load_shard(r, rows_per_tile):
            off = pl.multiple_of(sid * rows_per_tile, 8)
            pltpu.sync_copy(
                table_hbm.at[pl.ds(r * _SHARD + off, rows_per_tile)],
                shard_sp.at[pl.ds(off, rows_per_tile)],
            )

        def scat(slot):
            return pltpu.make_async_copy(
                rows_v.at[slot], out_hbm.at[pos_v.at[slot]], ssem[slot]
            )

        def do_chunk(q2, sub):
            slot = sub
            coff = (2 * q2 + sub) * _CHUNK
            for kk in range(_CHUNK // 16):
                pv = packed_v[pl.ds(coff + kk * 16, 16)]
                lrow_v[slot, pl.ds(kk * 16, 16)] = pv & _CAP
                pos_v[slot, pl.ds(kk * 16, 16)] = (pv >> 13) + base
            pltpu.make_async_copy(
                shard_sp.at[lrow_v.at[slot]], rows_v.at[slot], gsem
            ).start()
            pltpu.make_async_copy(
                shard_sp.at[lrow_v.at[slot]], rows_v.at[slot], gsem
            ).wait()
            scat(slot).start()

        def flush(off):
            nq2 = (off + 2 * _CHUNK - 1) // (2 * _CHUNK)

            @pl.when(off > 0)
            def _():
                dup = jnp.full((16,), packed_v[pl.ds(0, 16)][0], jnp.int32)
                t0 = (nq2 - 1) * 2 * _CHUNK
                for kk in range(2 * _CHUNK // 16):
                    g = pl.multiple_of(t0 + kk * 16, 16)
                    v = packed_v[pl.ds(g, 16)]
                    keep = (g + lane) < off
                    packed_v[pl.ds(g, 16)] = jnp.where(keep, v, dup)

            for q2 in range(max_pairs):
                @pl.when(q2 < nq2)
                def _():
                    @pl.when(done_ref[0] > 0)
                    def _():
                        scat(0).wait()
                        scat(1).wait()

                    do_chunk(q2, 0)
                    do_chunk(q2, 1)
                    done_ref[0] = 1

        def round_body(r):
            def seg_body(s, carry):
                def scan_body(i, off):
                    g = pl.multiple_of(s * _SEG + i * 16, 16)
                    v = idx_v[pl.ds(g, 16)]
                    m = (v >> 13) == r
                    packed = ((g + lane) << 13) | (v & _CAP)
                    mi = jnp.where(m, 1, 0)
                    ranks = mi
                    for sh in (1, 2, 4, 8):
                        prev = ranks.at[jnp.maximum(lane - sh, 0)].get(
                            mode="promise_in_bounds")
                        ranks = ranks + jnp.where(lane >= sh, prev, 0)
                    tgt = lane + 1
                    perm = jnp.zeros((16,), jnp.int32)
                    for sh in (8, 4, 2, 1):
                        probe = jnp.minimum(perm + sh - 1, 15)
                        rv = ranks.at[probe].get(mode="promise_in_bounds")
                        perm = jnp.where(rv < tgt, perm + sh, perm)
                    perm = jnp.minimum(perm, 15)
                    comp = packed.at[perm].get(mode="promise_in_bounds")
                    packed_v[pl.ds(pl.multiple_of(off, 8), 16)] = comp
                    return off + ranks[15]

                off = lax.fori_loop(0, _SEG // 16, scan_body, 0)
                flush(off)
                return carry

            lax.fori_loop(0, nseg, seg_body, 0)

        def full_round(kk, carry):
            r = 2 * kk + cid
            plsc.subcore_barrier()
            load_shard(r, _SHARD // _NS)
            plsc.subcore_barrier()
            round_body(r)
            return carry

        lax.fori_loop(0, n_full_pairs, full_round, 0)

        @pl.when(cid == (nshard - 1) % 2)
        def _():
            plsc.subcore_barrier()

            @pl.when(sid == 0)
            def _():
                pltpu.sync_copy(
                    table_hbm.at[pl.ds((nshard - 1) * _SHARD, last_rows)],
                    shard_sp.at[pl.ds(0, last_rows)],
                )

            plsc.subcore_barrier()
            round_body(nshard - 1)

        @pl.when(done_ref[0] > 0)
        def _():
            scat(0).wait()
            scat(1).wait()

    return k


def kernel(input_ids, embedding_weight):
    B, S = input_ids.shape
    V, D = embedding_weight.shape
    N = B * S
    idx = input_ids.reshape(N).astype(jnp.int32)
    out = _build_lookup(V, D, N)(embedding_weight, idx)
    return out.reshape(B, S, D)


# speedup vs baseline: 2.3344x; 2.3344x over previous
import functools

import jax
import jax.numpy as jnp
from jax import lax
from jax.experimental import pallas as pl
from jax.experimental.pallas import tpu as pltpu
from jax.experimental.pallas import tpu_sc as plsc

_NC = 2
_NS = 16
_NW = _NC * _NS


@functools.lru_cache(maxsize=None)
def _build_gather(V, D, N, C):
    assert N % _NW == 0
    b_per_w = N // _NW
    assert b_per_w % C == 0 and C % 8 == 0
    nchunks = b_per_w // C
    assert nchunks % 2 == 0

    NBUF = 5
    assert nchunks % NBUF == 0 and nchunks >= NBUF

    mesh = plsc.VectorSubcoreMesh(
        core_axis_name="c", subcore_axis_name="s",
        num_cores=_NC, num_subcores=_NS,
    )

    @functools.partial(
        pl.kernel,
        mesh=mesh,
        out_type=jax.ShapeDtypeStruct((N, D), jnp.float32),
        scratch_types=[
            pltpu.VMEM((b_per_w,), jnp.int32),
            pltpu.VMEM((NBUF, C, D), jnp.float32),
            [pltpu.SemaphoreType.DMA] * NBUF,
            [pltpu.SemaphoreType.DMA] * NBUF,
        ],
    )
    def k(table_hbm, idx_hbm, out_hbm, idx_v, rows_v, gsem, ssem):
        wid = lax.axis_index("s") * _NC + lax.axis_index("c")
        base = wid * b_per_w
        pltpu.sync_copy(idx_hbm.at[pl.ds(base, b_per_w)], idx_v)

        def gather(j, b):
            return pltpu.make_async_copy(
                table_hbm.at[idx_v.at[pl.ds(j * C, C)]], rows_v.at[b], gsem[b]
            )

        def issue(j, b):
            gather(j, b).start()

        def wait_gather(j, b):
            gather(j, b).wait()

        def store(j, b):
            off = base + j * C
            return pltpu.make_async_copy(
                rows_v.at[b], out_hbm.at[pl.ds(off, C)], ssem[b]
            )

        for b in range(NBUF - 1):
            issue(b, b)

        def ring_body(i, carry):
            for bb in range(NBUF):
                j = i * NBUF + bb
                jn = j + NBUF - 1
                bn = (bb + NBUF - 1) % NBUF

                @pl.when(jn < nchunks)
                def _():
                    @pl.when(jn >= NBUF)
                    def _():
                        store(jn - NBUF, bn).wait()

                    issue(jn, bn)

                wait_gather(j, bb)
                store(j, bb).start()
            return carry

        lax.fori_loop(0, nchunks // NBUF, ring_body, 0)

        for j in range(nchunks - NBUF, nchunks):
            store(j, j % NBUF).wait()

    return k


def kernel(input_ids, embedding_weight):
    B, S = input_ids.shape
    V, D = embedding_weight.shape
    N = B * S
    idx = input_ids.reshape(N).astype(jnp.int32)
    out = _build_gather(V, D, N, 128)(embedding_weight, idx)
    return out.reshape(B, S, D)
